# baseline (speedup 1.0000x reference)
import functools

import jax
import jax.numpy as jnp
from jax import lax
from jax.experimental import pallas as pl
from jax.experimental.pallas import tpu as pltpu
from jax.experimental.pallas import tpu_sc as plsc

_LANES = 16
_CHUNK = 128
_NBUF = 2


def _emb_body(n_chunks, n_per_w, scale,
              idxh_hbm, par_hbm, table_hbm, out_hbm,
              idxh_v, par_v, gbufs, sbufs, gsems, ssems):
    wid = lax.axis_index("s") * 2 + lax.axis_index("c")
    pltpu.sync_copy(idxh_hbm.at[pl.ds(wid * n_chunks, n_chunks)], idxh_v)
    pltpu.sync_copy(par_hbm.at[pl.ds(wid * n_chunks, n_chunks)], par_v)
    out_base = wid * (n_per_w // 2)
    lane = lax.iota(jnp.int32, _LANES)
    n_rounds = n_chunks // _NBUF

    def start_gather(g, b):
        pltpu.async_copy(table_hbm.at[idxh_v.at[g]], gbufs[b], gsems[b])

    def wait_gather(g, b):
        pltpu.make_async_copy(
            table_hbm.at[idxh_v.at[g]], gbufs[b], gsems[b]).wait()

    def out_slice(g):
        return out_hbm.at[pl.ds(out_base + g * (_CHUNK // 2), _CHUNK // 2)]

    def start_scatter(g, b):
        pltpu.async_copy(sbufs[b], out_slice(g), ssems[b])

    def wait_scatter(g, b):
        pltpu.make_async_copy(sbufs[b], out_slice(g), ssems[b]).wait()

    def scale_chunk(g, b):
        for k in range(_CHUNK // _LANES):
            r16 = k * _LANES + lane
            par16 = par_v[g, pl.ds(k * _LANES, _LANES)]
            gcol0 = par16 * 64
            srow = r16 >> 1
            scol0 = (r16 & 1) * 64

            def jbody(j, carry):
                v = plsc.load_gather(gbufs[b], [r16, gcol0 + j])
                plsc.store_scatter(sbufs[b], [srow, scol0 + j], v * scale)
                return carry

            lax.fori_loop(0, 64, jbody, 0, unroll=8)

    for b in range(_NBUF):
        start_gather(b, b)

    def round_body(t, carry):
        for b in range(_NBUF):
            g = t * _NBUF + b
            wait_gather(g, b)

            @pl.when(t > 0)
            def _():
                wait_scatter(g - _NBUF, b)

            scale_chunk(g, b)

            @pl.when(t < n_rounds - 1)
            def _():
                start_gather(g + _NBUF, b)

            start_scatter(g, b)
        return carry

    lax.fori_loop(0, n_rounds, round_body, 0)
    for b in range(_NBUF):
        wait_scatter((n_rounds - 1) * _NBUF + b, b)


def kernel(input_x, table):
    B0, S = input_x.shape
    V, D = table.shape
    B = B0 * S
    n_workers = 32
    n_per_w = B // n_workers
    n_chunks = n_per_w // _CHUNK
    scale = float(D) ** 0.5

    idx = input_x.reshape(B // _CHUNK, _CHUNK).astype(jnp.int32)
    idxh = idx >> 1
    par = idx & 1
    table128 = table.reshape(V // 2, 2 * D)

    mesh = plsc.VectorSubcoreMesh(core_axis_name="c", subcore_axis_name="s")
    emb = pl.kernel(
        functools.partial(_emb_body, n_chunks, n_per_w, scale),
        mesh=mesh,
        out_type=jax.ShapeDtypeStruct((B // 2, 2 * D), jnp.float32),
        scratch_types=[
            pltpu.VMEM((n_chunks, _CHUNK), jnp.int32),
            pltpu.VMEM((n_chunks, _CHUNK), jnp.int32),
            [pltpu.VMEM((_CHUNK, 2 * D), jnp.float32) for _ in range(_NBUF)],
            [pltpu.VMEM((_CHUNK // 2, 2 * D), jnp.float32)
             for _ in range(_NBUF)],
            [pltpu.SemaphoreType.DMA for _ in range(_NBUF)],
            [pltpu.SemaphoreType.DMA for _ in range(_NBUF)],
        ],
        compiler_params=pltpu.CompilerParams(
            use_tc_tiling_on_sc=True, needs_layout_passes=False),
    )
    out128 = emb(idxh, par, table128)
    return out128.reshape(B0, S, D)


# speedup vs baseline: 2.5813x; 2.5813x over previous
import functools

import jax
import jax.numpy as jnp
from jax import lax
from jax.experimental import pallas as pl
from jax.experimental.pallas import tpu as pltpu
from jax.experimental.pallas import tpu_sc as plsc

_LANES = 16
_CHUNK = 128
_NBUF = 2


def _emb_body(n_chunks, n_per_w, D, scale,
              idx_hbm, table_hbm, out_hbm, idx_v, gbufs, sbufs, gsems, ssems):
    wid = lax.axis_index("s") * 2 + lax.axis_index("c")
    pltpu.sync_copy(idx_hbm.at[pl.ds(wid * n_chunks, n_chunks)], idx_v)
    row_base = wid * n_per_w
    n_rounds = n_chunks // _NBUF
    scale_v = jnp.full((_LANES,), scale, dtype=jnp.float32)

    def start_gather(g, b):
        pltpu.async_copy(table_hbm.at[idx_v.at[g]], gbufs[b], gsems[b])

    def wait_gather(g, b):
        pltpu.make_async_copy(
            table_hbm.at[idx_v.at[g]], gbufs[b], gsems[b]).wait()

    def out_slice(g):
        return out_hbm.at[pl.ds(row_base + g * _CHUNK, _CHUNK)]

    def start_scatter(g, b):
        pltpu.async_copy(sbufs[b], out_slice(g), ssems[b])

    def wait_scatter(g, b):
        pltpu.make_async_copy(sbufs[b], out_slice(g), ssems[b]).wait()

    def scale_chunk(b):
        def row_body(r, carry):
            for c in range(D // _LANES):
                sbufs[b][r, pl.ds(c * _LANES, _LANES)] = (
                    gbufs[b][r, pl.ds(c * _LANES, _LANES)] * scale_v)
            return carry

        lax.fori_loop(0, _CHUNK, row_body, 0, unroll=2)

    for b in range(_NBUF):
        start_gather(b, b)

    def round_body(t, carry):
        for b in range(_NBUF):
            g = t * _NBUF + b
            wait_gather(g, b)

            @pl.when(t > 0)
            def _():
                wait_scatter(g - _NBUF, b)

            scale_chunk(b)

            @pl.when(t < n_rounds - 1)
            def _():
                start_gather(g + _NBUF, b)

            start_scatter(g, b)
        return carry

    lax.fori_loop(0, n_rounds, round_body, 0)
    for b in range(_NBUF):
        wait_scatter((n_rounds - 1) * _NBUF + b, b)


def kernel(input_x, table):
    B0, S = input_x.shape
    V, D = table.shape
    B = B0 * S
    n_workers = 32
    n_per_w = B // n_workers
    n_chunks = n_per_w // _CHUNK
    scale = float(D) ** 0.5

    idx2d = input_x.reshape(B // _CHUNK, _CHUNK).astype(jnp.int32)
    table_pad = jnp.pad(table, ((0, 0), (0, 128 - D)))

    mesh = plsc.VectorSubcoreMesh(core_axis_name="c", subcore_axis_name="s")
    emb = pl.kernel(
        functools.partial(_emb_body, n_chunks, n_per_w, D, scale),
        mesh=mesh,
        out_type=jax.ShapeDtypeStruct((B, D), jnp.float32),
        scratch_types=[
            pltpu.VMEM((n_chunks, _CHUNK), jnp.int32),
            [pltpu.VMEM((_CHUNK, 128), jnp.float32) for _ in range(_NBUF)],
            [pltpu.VMEM((_CHUNK, D), jnp.float32) for _ in range(_NBUF)],
            [pltpu.SemaphoreType.DMA for _ in range(_NBUF)],
            [pltpu.SemaphoreType.DMA for _ in range(_NBUF)],
        ],
        compiler_params=pltpu.CompilerParams(use_tc_tiling_on_sc=True),
    )
    out = emb(idx2d, table_pad)
    return out.reshape(B0, S, D)
